# Initial kernel scaffold; baseline (speedup 1.0000x reference)
import jax
import jax.numpy as jnp
from jax.experimental import pallas as pl


def kernel(input_ids, position_ids, pos_table, ln_weight, ln_bias):
    raise NotImplementedError("write your pallas kernel here")


# speedup vs baseline: 19.7576x; 19.7576x over previous
import functools

import jax
import jax.numpy as jnp
from jax import lax
from jax.experimental import pallas as pl
from jax.experimental.pallas import tpu as pltpu
from jax.experimental.pallas import tpu_sc as plsc

LN_EPS = 1e-5


def _ln_table_body(tab_ref, w_ref, b_ref, out_ref):
    x = tab_ref[...]
    mean = jnp.mean(x, axis=1, keepdims=True)
    var = jnp.mean((x - mean) ** 2, axis=1, keepdims=True)
    normed = (x - mean) / jnp.sqrt(var + LN_EPS)
    out_ref[...] = normed * w_ref[...] + b_ref[...]


def _ln_table(pos_table, ln_weight, ln_bias):
    v, d = pos_table.shape
    return pl.pallas_call(
        _ln_table_body,
        out_shape=jax.ShapeDtypeStruct((v, d), jnp.float32),
    )(pos_table, ln_weight.reshape(1, d), ln_bias.reshape(1, d))


_CHUNK = 128


def _make_gather(n_rows, d, v):
    info = plsc.get_sparse_core_info()
    nw = info.num_cores * info.num_subcores
    assert n_rows % (nw * _CHUNK) == 0
    cw = n_rows // (nw * _CHUNK)
    assert cw % 4 == 0
    mesh = plsc.VectorSubcoreMesh(core_axis_name="c", subcore_axis_name="s")

    @functools.partial(
        pl.kernel,
        mesh=mesh,
        out_type=jax.ShapeDtypeStruct((n_rows, d), jnp.float32),
        scratch_types=(
            [pltpu.VMEM((cw, _CHUNK), jnp.int32),
             pltpu.VMEM_SHARED((v, d), jnp.float32)]
            + [pltpu.VMEM((_CHUNK, d), jnp.float32)] * 4
            + [pltpu.SemaphoreType.DMA] * 8
        ),
    )
    def gather(tab_hbm, idx_hbm, out_hbm, idx_v, tab_sh, r0, r1, r2, r3,
               g0, g1, g2, g3, w0, w1, w2, w3):
        sid = lax.axis_index("s")
        wid = sid * info.num_cores + lax.axis_index("c")
        base = wid * cw

        @pl.when(sid == 0)
        def _():
            pltpu.sync_copy(tab_hbm, tab_sh)

        pltpu.sync_copy(idx_hbm.at[pl.ds(base, cw)], idx_v)
        plsc.subcore_barrier()

        bufs = ((r0, g0, w0), (r1, g1, w1), (r2, g2, w2), (r3, g3, w3))

        def start_gather(j, b):
            rows, gsem, _ = bufs[b]
            pltpu.async_copy(tab_sh.at[idx_v.at[j]], rows, gsem)

        def wait_gather(b):
            rows, gsem, _ = bufs[b]
            pltpu.make_async_copy(tab_hbm.at[pl.ds(0, _CHUNK)], rows,
                                  gsem).wait()

        def start_write(j, b):
            rows, _, wsem = bufs[b]
            pltpu.async_copy(rows, out_hbm.at[pl.ds((base + j) * _CHUNK,
                                                    _CHUNK)], wsem)

        def wait_write(b):
            rows, _, wsem = bufs[b]
            pltpu.make_async_copy(rows, out_hbm.at[pl.ds(0, _CHUNK)],
                                  wsem).wait()

        start_gather(0, 0)
        start_gather(1, 1)
        start_gather(2, 2)

        def body(g, _):
            for b in range(4):
                j = 4 * g + b
                wait_gather(b)
                start_write(j, b)
                bn = (b + 3) % 4

                @pl.when(j + 3 < cw)
                def _():
                    @pl.when(j >= 1)
                    def _():
                        wait_write(bn)
                    start_gather(j + 3, bn)
            return 0

        lax.fori_loop(0, cw // 4, body, 0)
        for b in range(4):
            wait_write(b)

    return gather


def kernel(input_ids, position_ids, pos_table, ln_weight, ln_bias):
    b, s = position_ids.shape
    v, d = pos_table.shape
    n = b * s
    normed = _ln_table(pos_table, ln_weight, ln_bias)
    idx = position_ids.reshape(n // _CHUNK, _CHUNK).astype(jnp.int32)
    out = _make_gather(n, d, v)(normed, idx)
    return out.reshape(b, s, d)
